# baseline (speedup 1.0000x reference)
import functools

import jax
import jax.numpy as jnp
from jax import lax
from jax.experimental import pallas as pl
from jax.experimental.pallas import tpu as pltpu
from jax.experimental.pallas import tpu_sc as plsc

VOCAB = 1000000
HIDDEN = 64
BATCH = 16384

NC = 2
NS = 16
NW = NC * NS
BPW = BATCH // NW
G = 16
SROW = 128
VPAD = 1000448
NSROW = VPAD // SROW

_DN = (((0,), (0,)), ((), ()))
_BLKV = 2048


def _k1_body(tabT_ref, w1_ref, b1_ref, w2_ref, b2_ref, out_ref):
    h = lax.dot_general(w1_ref[...], tabT_ref[...], _DN,
                        preferred_element_type=jnp.float32)
    h = jnp.maximum(h + b1_ref[...], 0.0)
    logits = lax.dot_general(w2_ref[...], h, _DN,
                             preferred_element_type=jnp.float32)
    out_ref[...] = jax.nn.sigmoid(logits + b2_ref[...])


def _tc_vocab_eval(tabT, W1, b1, W2, b2):
    grid = ((VOCAB + _BLKV - 1) // _BLKV,)
    return pl.pallas_call(
        _k1_body,
        grid=grid,
        in_specs=[
            pl.BlockSpec((HIDDEN, _BLKV), lambda i: (0, i)),
            pl.BlockSpec((HIDDEN, 128), lambda i: (0, 0)),
            pl.BlockSpec((128, 1), lambda i: (0, 0)),
            pl.BlockSpec((128, 1), lambda i: (0, 0)),
            pl.BlockSpec((1, 1), lambda i: (0, 0)),
        ],
        out_specs=pl.BlockSpec((1, _BLKV), lambda i: (0, i)),
        out_shape=jax.ShapeDtypeStruct((1, VOCAB), jnp.float32),
    )(tabT, W1, b1, W2, b2)


@functools.lru_cache(maxsize=1)
def _sc_gather_build():
    mesh = plsc.VectorSubcoreMesh(core_axis_name="c", subcore_axis_name="s")

    @functools.partial(
        pl.kernel,
        mesh=mesh,
        out_type=jax.ShapeDtypeStruct((BATCH, SROW), jnp.float32),
        scratch_types=[
            pltpu.VMEM((BPW,), jnp.int32),
            pltpu.VMEM((BPW, SROW), jnp.float32),
            pltpu.SemaphoreType.DMA,
        ],
    )
    def gather_kernel(sig_hbm, tok_hbm, out_hbm, tok_v, rows_v, sem):
        wid = lax.axis_index("s") * NC + lax.axis_index("c")
        pltpu.sync_copy(tok_hbm.at[wid], tok_v)

        def fire_group(g):
            v16 = tok_v[pl.ds(g * G, G)]
            for j in range(G):
                r = v16[j] >> 7
                pltpu.async_copy(
                    sig_hbm.at[r >> 3, r & 7],
                    rows_v.at[g * G + j],
                    sem,
                )

        def drain_group(g):
            pltpu.make_async_copy(
                out_hbm.at[pl.ds(0, G)],
                rows_v.at[pl.ds(g * G, G)],
                sem,
            ).wait()

        def head(g, _):
            fire_group(g)
            return 0

        def pipelined(g, _):
            fire_group(g)
            drain_group(g - 1)
            return 0

        lax.fori_loop(0, 1, head, 0)
        lax.fori_loop(1, BPW // G, pipelined, 0)
        drain_group(BPW // G - 1)
        pltpu.sync_copy(rows_v, out_hbm.at[pl.ds(wid * BPW, BPW)])

    return gather_kernel


_BLK = 4096


def _k2_body(vals_ref, tmod_ref, out_ref):
    lane = lax.broadcasted_iota(jnp.int32, (1, SROW), 1)
    sel = jnp.where(tmod_ref[...] == lane, vals_ref[...], 0.0)
    out_ref[...] = jnp.sum(sel, axis=1, keepdims=True)


def _tc_select(vals, tmod):
    grid = (BATCH // _BLK,)
    return pl.pallas_call(
        _k2_body,
        grid=grid,
        in_specs=[
            pl.BlockSpec((_BLK, SROW), lambda i: (i, 0)),
            pl.BlockSpec((_BLK, 1), lambda i: (i, 0)),
        ],
        out_specs=pl.BlockSpec((_BLK, 1), lambda i: (i, 0)),
        out_shape=jax.ShapeDtypeStruct((BATCH, 1), jnp.float32),
    )(vals, tmod)


def kernel(tool_token, table, W1, b1, W2, b2):
    tok = tool_token.astype(jnp.int32)
    tok2 = tok.reshape(NW, BPW)
    tmod = (tok & (SROW - 1)).reshape(BATCH, 1)
    tabT = table.T
    sig_row = _tc_vocab_eval(tabT, W1, b1.reshape(128, 1), W2,
                             b2.reshape(1, 1))
    sigp = jnp.pad(sig_row, ((0, 0), (0, VPAD - VOCAB)))
    sig3 = sigp.reshape(NSROW // 8, 8, SROW)
    vals = _sc_gather_build()(sig3, tok2)
    return _tc_select(vals, tmod)


# speedup vs baseline: 1.9911x; 1.9911x over previous
import functools

import jax
import jax.numpy as jnp
from jax import lax
from jax.experimental import pallas as pl
from jax.experimental.pallas import tpu as pltpu
from jax.experimental.pallas import tpu_sc as plsc

VOCAB = 1000000
HIDDEN = 64
BATCH = 16384

NC = 2
NS = 16
NW = NC * NS
BPW = BATCH // NW
G = 16
SROW = 128
VPAD = 1000448
NSROW = VPAD // SROW

_DN = (((0,), (0,)), ((), ()))
_BLKV = 8192


def _k1_body(tabT_ref, w1_ref, b1_ref, w2_ref, b2_ref, out_ref):
    h = lax.dot_general(w1_ref[...], tabT_ref[...], _DN,
                        preferred_element_type=jnp.float32)
    h = jnp.maximum(h + b1_ref[...], 0.0)
    logits = lax.dot_general(w2_ref[...], h, _DN,
                             preferred_element_type=jnp.float32)
    out_ref[...] = jax.nn.sigmoid(logits + b2_ref[...])


def _tc_vocab_eval(tabT, W1, b1, W2, b2):
    grid = ((VOCAB + _BLKV - 1) // _BLKV,)
    return pl.pallas_call(
        _k1_body,
        grid=grid,
        in_specs=[
            pl.BlockSpec((HIDDEN, _BLKV), lambda i: (0, i)),
            pl.BlockSpec((HIDDEN, 128), lambda i: (0, 0)),
            pl.BlockSpec((128, 1), lambda i: (0, 0)),
            pl.BlockSpec((128, 1), lambda i: (0, 0)),
            pl.BlockSpec((1, 1), lambda i: (0, 0)),
        ],
        out_specs=pl.BlockSpec((1, _BLKV), lambda i: (0, i)),
        out_shape=jax.ShapeDtypeStruct((1, VOCAB), jnp.float32),
    )(tabT, W1, b1, W2, b2)


@functools.lru_cache(maxsize=1)
def _sc_gather_build():
    mesh = plsc.VectorSubcoreMesh(core_axis_name="c", subcore_axis_name="s")

    @functools.partial(
        pl.kernel,
        mesh=mesh,
        out_type=jax.ShapeDtypeStruct((BATCH, SROW), jnp.float32),
        scratch_types=[
            pltpu.VMEM((BPW,), jnp.int32),
            pltpu.VMEM((BPW, SROW), jnp.float32),
            pltpu.SemaphoreType.DMA,
        ],
    )
    def gather_kernel(sig_hbm, tok_hbm, out_hbm, tok_v, rows_v, sem):
        wid = lax.axis_index("s") * NC + lax.axis_index("c")
        pltpu.sync_copy(tok_hbm.at[wid], tok_v)

        def fire_group(g):
            v16 = tok_v[pl.ds(g * G, G)]
            for j in range(G):
                r = v16[j] >> 7
                pltpu.async_copy(
                    sig_hbm.at[r >> 3, r & 7],
                    rows_v.at[g * G + j],
                    sem,
                )

        def drain_group(g):
            pltpu.make_async_copy(
                out_hbm.at[pl.ds(0, G)],
                rows_v.at[pl.ds(g * G, G)],
                sem,
            ).wait()

        def head(g, _):
            fire_group(g)
            return 0

        def pipelined(g, _):
            fire_group(g)
            drain_group(g - 1)
            return 0

        lax.fori_loop(0, 1, head, 0)
        lax.fori_loop(1, BPW // G, pipelined, 0)
        drain_group(BPW // G - 1)
        pltpu.sync_copy(rows_v, out_hbm.at[pl.ds(wid * BPW, BPW)])

    return gather_kernel


_BLK = 4096


def _k2_body(vals_ref, tmod_ref, out_ref):
    lane = lax.broadcasted_iota(jnp.int32, (1, SROW), 1)
    sel = jnp.where(tmod_ref[...] == lane, vals_ref[...], 0.0)
    out_ref[...] = jnp.sum(sel, axis=1, keepdims=True)


def _tc_select(vals, tmod):
    grid = (BATCH // _BLK,)
    return pl.pallas_call(
        _k2_body,
        grid=grid,
        in_specs=[
            pl.BlockSpec((_BLK, SROW), lambda i: (i, 0)),
            pl.BlockSpec((_BLK, 1), lambda i: (i, 0)),
        ],
        out_specs=pl.BlockSpec((_BLK, 1), lambda i: (i, 0)),
        out_shape=jax.ShapeDtypeStruct((BATCH, 1), jnp.float32),
    )(vals, tmod)


def kernel(tool_token, table, W1, b1, W2, b2):
    tok = tool_token.astype(jnp.int32)
    tok2 = tok.reshape(NW, BPW)
    tmod = (tok & (SROW - 1)).reshape(BATCH, 1)
    tabT = table.T
    sig_row = _tc_vocab_eval(tabT, W1, b1.reshape(128, 1), W2,
                             b2.reshape(1, 1))
    sigp = jnp.pad(sig_row, ((0, 0), (0, VPAD - VOCAB)))
    sig3 = sigp.reshape(NSROW // 8, 8, SROW)
    vals = _sc_gather_build()(sig3, tok2)
    return _tc_select(vals, tmod)
